# baseline (speedup 1.0000x reference)
import functools

import jax
import jax.numpy as jnp
from jax import lax
from jax.experimental import pallas as pl
from jax.experimental.pallas import tpu as pltpu
from jax.experimental.pallas import tpu_sc as plsc

_D = 64
_K = 1024
_B = 32
_R = 1024
_N = _B * _R
_NW = 32
_BW = _N // _NW


def _vq_body(xt_ref, et_ref, idx_ref, lin_ref, tab_ref, perp_ref, w_ref,
             counts_ref):
    i = pl.program_id(0)
    xt = xt_ref[0]
    et = et_ref[...]

    @pl.when(i == 0)
    def _mkw():
        r = jax.lax.broadcasted_iota(jnp.int32, (8, _K), 0)
        c = jax.lax.broadcasted_iota(jnp.int32, (8, _K), 1)
        w = jnp.where(r == 0, c >> 5,
                      jnp.where(r == 1, c & 31,
                                jnp.where(r == 2, 1, 0)))
        w_ref[...] = w.astype(jnp.float32)
        tab_ref[:, 0:_D] = et.T
        tab_ref[:, _D:128] = jnp.zeros((_K, _D), jnp.float32)

    a_sq = jnp.sum(xt * xt, axis=0, keepdims=True)
    ab = 2.0 * jax.lax.dot_general(
        et, xt, (((0,), (0,)), ((), ())),
        preferred_element_type=jnp.float32)
    ones8 = jnp.ones((_D, 8), jnp.float32)
    b_sq = jax.lax.dot_general(
        et * et, ones8, (((0,), (0,)), ((), ())),
        precision=jax.lax.Precision.HIGHEST,
        preferred_element_type=jnp.float32)[:, 0:1]
    dist = (a_sq - ab) + b_sq

    dmin = jnp.min(dist, axis=0, keepdims=True)
    eqf = (dist == dmin).astype(jnp.float32)
    mm = jax.lax.dot_general(
        w_ref[...], eqf, (((1,), (0,)), ((), ())),
        preferred_element_type=jnp.float32)
    hi = mm[0:1, :]
    lo = mm[1:2, :]
    cnt = mm[2:3, :]
    tie = jnp.max(cnt) > 1.5

    def _store_idx(row):
        t = jnp.broadcast_to(row.reshape(1, _K), (_B, _K))
        rows = jax.lax.broadcasted_iota(jnp.int32, (_B, _K), 0)
        prev = jnp.where(i == 0, jnp.zeros((_B, _K), jnp.int32), idx_ref[...])
        idx_ref[...] = jnp.where(rows == i, t, prev)
        lin_ref[...] = row.reshape(8, 128)

    _store_idx((hi * 32.0 + lo)[0, :].astype(jnp.int32))

    blk_counts = jnp.sum(eqf, axis=1, keepdims=True)

    @pl.when(i == 0)
    def _init():
        counts_ref[...] = blk_counts

    @pl.when(i > 0)
    def _acc():
        counts_ref[...] += blk_counts

    @pl.when(tie)
    def _slow():
        ids = jax.lax.broadcasted_iota(jnp.int32, (_K, _R), 0)
        idxv = jnp.min(jnp.where(dist == dmin, ids, _K), axis=0,
                       keepdims=True)
        oh = (ids == idxv).astype(jnp.float32)
        _store_idx(idxv[0, :])
        counts_ref[...] += jnp.sum(oh, axis=1, keepdims=True) - blk_counts

    @pl.when(i == _B - 1)
    def _final():
        p = counts_ref[...] * (1.0 / _N)
        ent = -jnp.sum(p * jnp.log(p + 1e-10))
        perp_ref[0, 0] = jnp.exp(ent)


@functools.cache
def _sc_gather_fn():
    mesh = plsc.VectorSubcoreMesh(core_axis_name="c", subcore_axis_name="s")

    @functools.partial(
        pl.kernel,
        out_type=jax.ShapeDtypeStruct((_N, 128), jnp.float32),
        mesh=mesh,
        scratch_types=[
            pltpu.VMEM((_BW,), jnp.int32),
            pltpu.VMEM((_BW // 2, 128), jnp.float32),
            pltpu.SemaphoreType.DMA,
        ],
    )
    def _sc_gather(tab_hbm, idx_hbm, out_hbm, idx_v, rows_v, sem):
        wid = lax.axis_index("s") * 2 + lax.axis_index("c")
        base = wid * _BW
        pltpu.sync_copy(idx_hbm.at[pl.ds(base, _BW)], idx_v)
        for h in range(2):
            pltpu.async_copy(
                tab_hbm.at[idx_v.at[pl.ds(h * (_BW // 2), _BW // 2)]],
                rows_v, sem).wait()
            pltpu.sync_copy(rows_v,
                            out_hbm.at[pl.ds(base + h * (_BW // 2), _BW // 2)])

    return _sc_gather


def kernel(inputs, embeddings):
    xt = jnp.transpose(inputs, (0, 2, 1))
    et = embeddings.T
    idx, lin, tab, perp = pl.pallas_call(
        _vq_body,
        grid=(_B,),
        in_specs=[
            pl.BlockSpec((1, _D, _R), lambda i: (i, 0, 0)),
            pl.BlockSpec((_D, _K), lambda i: (0, 0)),
        ],
        out_specs=[
            pl.BlockSpec((_B, _K), lambda i: (0, 0)),
            pl.BlockSpec((8, 128), lambda i: (i, 0)),
            pl.BlockSpec((_K, 128), lambda i: (0, 0)),
            pl.BlockSpec(memory_space=pltpu.SMEM),
        ],
        out_shape=[
            jax.ShapeDtypeStruct((_B, _K), jnp.int32),
            jax.ShapeDtypeStruct((_B * 8, 128), jnp.int32),
            jax.ShapeDtypeStruct((_K, 128), jnp.float32),
            jax.ShapeDtypeStruct((1, 1), jnp.float32),
        ],
        scratch_shapes=[
            pltpu.VMEM((8, _K), jnp.float32),
            pltpu.VMEM((_K, 1), jnp.float32),
        ],
    )(xt, et)
    q = _sc_gather_fn()(tab, lin.reshape(_N))
    quantized_st = q[:, 0:_D].reshape(inputs.shape)
    return (quantized_st, idx, perp[0, 0])


# speedup vs baseline: 1.8707x; 1.8707x over previous
import jax
import jax.numpy as jnp
from jax import lax
from jax.experimental import pallas as pl
from jax.experimental.pallas import tpu as pltpu

_D = 64
_K = 1024
_B = 32
_R = 1024
_BB = 2
_NB = _B // _BB
_G = 72


def _vq_body(xt_ref, et_ref, qt_ref, idx_ref, perp_ref, g_ref, counts_ref):
    i = pl.program_id(0)
    et = et_ref[...]

    @pl.when(i == 0)
    def _mkg():
        r = jax.lax.broadcasted_iota(jnp.int32, (_G - _D, _K), 0)
        c = jax.lax.broadcasted_iota(jnp.int32, (_G - _D, _K), 1)
        w = jnp.where(r == 0, c >> 5,
                      jnp.where(r == 1, c & 31,
                                jnp.where(r == 2, 1, 0)))
        g_ref[0:_D, :] = et
        g_ref[_D:_G, :] = w.astype(jnp.float32)

    ones8 = jnp.ones((_D, 8), jnp.float32)
    b_sq = jax.lax.dot_general(
        et * et, ones8, (((0,), (0,)), ((), ())),
        precision=jax.lax.Precision.HIGHEST,
        preferred_element_type=jnp.float32)[:, 0:1]

    idx_rows = []
    step_counts = None
    for j in range(_BB):
        xt = xt_ref[j]
        a_sq = jnp.sum(xt * xt, axis=0, keepdims=True)
        ab = 2.0 * jax.lax.dot_general(
            et, xt, (((0,), (0,)), ((), ())),
            preferred_element_type=jnp.float32)
        dist = (a_sq - ab) + b_sq

        dmin = jnp.min(dist, axis=0, keepdims=True)
        eqf = (dist == dmin).astype(jnp.float32)
        mm = jax.lax.dot_general(
            g_ref[...], eqf, (((1,), (0,)), ((), ())),
            preferred_element_type=jnp.float32)
        tie = jnp.max(mm[_D + 2:_D + 3, :]) > 1.5

        def _fast(mm=mm, eqf=eqf):
            idx = (mm[_D:_D + 1, :] * 32.0
                   + mm[_D + 1:_D + 2, :])[0, :].astype(jnp.int32)
            return (idx, mm[0:_D, :], jnp.sum(eqf, axis=1, keepdims=True))

        def _slow(dist=dist, dmin=dmin):
            ids = jax.lax.broadcasted_iota(jnp.int32, (_K, _R), 0)
            idxv = jnp.min(jnp.where(dist == dmin, ids, _K), axis=0,
                           keepdims=True)
            oh = (ids == idxv).astype(jnp.float32)
            q2 = jax.lax.dot_general(
                et, oh, (((1,), (0,)), ((), ())),
                preferred_element_type=jnp.float32)
            return (idxv[0, :], q2, jnp.sum(oh, axis=1, keepdims=True))

        idx_j, qt_j, counts_j = lax.cond(tie, _slow, _fast)
        qt_ref[j] = xt + (qt_j - xt)
        idx_rows.append(idx_j)
        step_counts = counts_j if step_counts is None else step_counts + counts_j

    def _store_idx(rows_block):
        t = jnp.broadcast_to(rows_block.reshape(1, _BB, _K),
                             (_NB, _BB, _K)).reshape(_B, _K)
        rows = jax.lax.broadcasted_iota(jnp.int32, (_B, _K), 0)
        prev = jnp.where(i == 0, jnp.zeros((_B, _K), jnp.int32), idx_ref[...])
        idx_ref[...] = jnp.where((rows >> 1) == i, t, prev)

    _store_idx(jnp.stack(idx_rows, axis=0))

    @pl.when(i == 0)
    def _init():
        counts_ref[...] = step_counts

    @pl.when(i > 0)
    def _acc():
        counts_ref[...] += step_counts

    @pl.when(i == _NB - 1)
    def _final():
        p = counts_ref[...] * (1.0 / (_B * _R))
        ent = -jnp.sum(p * jnp.log(p + 1e-10))
        perp_ref[0, 0] = jnp.exp(ent)


def kernel(inputs, embeddings):
    xt = jnp.transpose(inputs, (0, 2, 1))
    et = embeddings.T
    qt, idx, perp = pl.pallas_call(
        _vq_body,
        grid=(_NB,),
        in_specs=[
            pl.BlockSpec((_BB, _D, _R), lambda i: (i, 0, 0)),
            pl.BlockSpec((_D, _K), lambda i: (0, 0)),
        ],
        out_specs=[
            pl.BlockSpec((_BB, _D, _R), lambda i: (i, 0, 0)),
            pl.BlockSpec((_B, _K), lambda i: (0, 0)),
            pl.BlockSpec(memory_space=pltpu.SMEM),
        ],
        out_shape=[
            jax.ShapeDtypeStruct((_B, _D, _R), jnp.float32),
            jax.ShapeDtypeStruct((_B, _K), jnp.int32),
            jax.ShapeDtypeStruct((1, 1), jnp.float32),
        ],
        scratch_shapes=[
            pltpu.VMEM((_G, _K), jnp.float32),
            pltpu.VMEM((_K, 1), jnp.float32),
        ],
    )(xt, et)
    quantized_st = jnp.transpose(qt, (0, 2, 1))
    return (quantized_st, idx, perp[0, 0])


# speedup vs baseline: 2.0311x; 1.0857x over previous
import jax
import jax.numpy as jnp
from jax import lax
from jax.experimental import pallas as pl
from jax.experimental.pallas import tpu as pltpu

_D = 64
_K = 1024
_B = 32
_R = 1024
_BB = 4
_NB = _B // _BB
_G = 72


def _vq_body(xt_ref, et_ref, qt_ref, idx_ref, perp_ref, g_ref, counts_ref):
    i = pl.program_id(0)
    et = et_ref[...]

    @pl.when(i == 0)
    def _mkg():
        r = jax.lax.broadcasted_iota(jnp.int32, (_G - _D, _K), 0)
        c = jax.lax.broadcasted_iota(jnp.int32, (_G - _D, _K), 1)
        w = jnp.where(r == 0, c >> 5,
                      jnp.where(r == 1, c & 31,
                                jnp.where(r == 2, 1, 0)))
        g_ref[0:_D, :] = et
        g_ref[_D:_G, :] = w.astype(jnp.float32)

    ones8 = jnp.ones((_D, 8), jnp.float32)
    b_sq = jax.lax.dot_general(
        et * et, ones8, (((0,), (0,)), ((), ())),
        precision=jax.lax.Precision.HIGHEST,
        preferred_element_type=jnp.float32)[:, 0:1]

    idx_rows = []
    step_counts = None
    for j in range(_BB):
        xt = xt_ref[j]
        a_sq = jnp.sum(xt * xt, axis=0, keepdims=True)
        ab = 2.0 * jax.lax.dot_general(
            et, xt, (((0,), (0,)), ((), ())),
            preferred_element_type=jnp.float32)
        dist = (a_sq - ab) + b_sq

        dmin = jnp.min(dist, axis=0, keepdims=True)
        eqf = (dist == dmin).astype(jnp.float32)
        mm = jax.lax.dot_general(
            g_ref[...], eqf, (((1,), (0,)), ((), ())),
            preferred_element_type=jnp.float32)
        tie = jnp.max(mm[_D + 2:_D + 3, :]) > 1.5

        def _fast(mm=mm, eqf=eqf):
            idx = (mm[_D:_D + 1, :] * 32.0
                   + mm[_D + 1:_D + 2, :])[0, :].astype(jnp.int32)
            return (idx, mm[0:_D, :], jnp.sum(eqf, axis=1, keepdims=True))

        def _slow(dist=dist, dmin=dmin):
            ids = jax.lax.broadcasted_iota(jnp.int32, (_K, _R), 0)
            idxv = jnp.min(jnp.where(dist == dmin, ids, _K), axis=0,
                           keepdims=True)
            oh = (ids == idxv).astype(jnp.float32)
            q2 = jax.lax.dot_general(
                et, oh, (((1,), (0,)), ((), ())),
                preferred_element_type=jnp.float32)
            return (idxv[0, :], q2, jnp.sum(oh, axis=1, keepdims=True))

        idx_j, qt_j, counts_j = lax.cond(tie, _slow, _fast)
        qt_ref[j] = xt + (qt_j - xt)
        idx_rows.append(idx_j)
        step_counts = counts_j if step_counts is None else step_counts + counts_j

    def _store_idx(rows_block):
        t = jnp.broadcast_to(rows_block.reshape(1, _BB, _K),
                             (_NB, _BB, _K)).reshape(_B, _K)
        rows = jax.lax.broadcasted_iota(jnp.int32, (_B, _K), 0)
        prev = jnp.where(i == 0, jnp.zeros((_B, _K), jnp.int32), idx_ref[...])
        idx_ref[...] = jnp.where((rows >> 2) == i, t, prev)

    _store_idx(jnp.stack(idx_rows, axis=0))

    @pl.when(i == 0)
    def _init():
        counts_ref[...] = step_counts

    @pl.when(i > 0)
    def _acc():
        counts_ref[...] += step_counts

    @pl.when(i == _NB - 1)
    def _final():
        p = counts_ref[...] * (1.0 / (_B * _R))
        ent = -jnp.sum(p * jnp.log(p + 1e-10))
        perp_ref[0, 0] = jnp.exp(ent)


def kernel(inputs, embeddings):
    xt = jnp.transpose(inputs, (0, 2, 1))
    et = embeddings.T
    qt, idx, perp = pl.pallas_call(
        _vq_body,
        grid=(_NB,),
        in_specs=[
            pl.BlockSpec((_BB, _D, _R), lambda i: (i, 0, 0)),
            pl.BlockSpec((_D, _K), lambda i: (0, 0)),
        ],
        out_specs=[
            pl.BlockSpec((_BB, _D, _R), lambda i: (i, 0, 0)),
            pl.BlockSpec((_B, _K), lambda i: (0, 0)),
            pl.BlockSpec(memory_space=pltpu.SMEM),
        ],
        out_shape=[
            jax.ShapeDtypeStruct((_B, _D, _R), jnp.float32),
            jax.ShapeDtypeStruct((_B, _K), jnp.int32),
            jax.ShapeDtypeStruct((1, 1), jnp.float32),
        ],
        scratch_shapes=[
            pltpu.VMEM((_G, _K), jnp.float32),
            pltpu.VMEM((_K, 1), jnp.float32),
        ],
    )(xt, et)
    quantized_st = jnp.transpose(qt, (0, 2, 1))
    return (quantized_st, idx, perp[0, 0])


# speedup vs baseline: 2.1170x; 1.0423x over previous
import jax
import jax.numpy as jnp
from jax import lax
from jax.experimental import pallas as pl
from jax.experimental.pallas import tpu as pltpu

_D = 64
_K = 1024
_B = 32
_R = 1024
_BB = 8
_NB = _B // _BB
_G = 72


def _vq_body(xt_ref, et_ref, qt_ref, idx_ref, perp_ref, g_ref, counts_ref):
    i = pl.program_id(0)
    et = et_ref[...]

    @pl.when(i == 0)
    def _mkg():
        r = jax.lax.broadcasted_iota(jnp.int32, (_G - _D, _K), 0)
        c = jax.lax.broadcasted_iota(jnp.int32, (_G - _D, _K), 1)
        w = jnp.where(r == 0, c >> 5,
                      jnp.where(r == 1, c & 31,
                                jnp.where(r == 2, 1, 0)))
        g_ref[0:_D, :] = et
        g_ref[_D:_G, :] = w.astype(jnp.float32)

    ones8 = jnp.ones((_D, 8), jnp.float32)
    b_sq = jax.lax.dot_general(
        et * et, ones8, (((0,), (0,)), ((), ())),
        precision=jax.lax.Precision.HIGHEST,
        preferred_element_type=jnp.float32)[:, 0:1]

    idx_rows = []
    step_counts = None
    for j in range(_BB):
        xt = xt_ref[j]
        a_sq = jnp.sum(xt * xt, axis=0, keepdims=True)
        ab = 2.0 * jax.lax.dot_general(
            et, xt, (((0,), (0,)), ((), ())),
            preferred_element_type=jnp.float32)
        dist = (a_sq - ab) + b_sq

        dmin = jnp.min(dist, axis=0, keepdims=True)
        eqf = (dist == dmin).astype(jnp.float32)
        mm = jax.lax.dot_general(
            g_ref[...], eqf, (((1,), (0,)), ((), ())),
            preferred_element_type=jnp.float32)
        tie = jnp.max(mm[_D + 2:_D + 3, :]) > 1.5

        def _fast(mm=mm, eqf=eqf):
            idx = (mm[_D:_D + 1, :] * 32.0
                   + mm[_D + 1:_D + 2, :])[0, :].astype(jnp.int32)
            return (idx, mm[0:_D, :], jnp.sum(eqf, axis=1, keepdims=True))

        def _slow(dist=dist, dmin=dmin):
            ids = jax.lax.broadcasted_iota(jnp.int32, (_K, _R), 0)
            idxv = jnp.min(jnp.where(dist == dmin, ids, _K), axis=0,
                           keepdims=True)
            oh = (ids == idxv).astype(jnp.float32)
            q2 = jax.lax.dot_general(
                et, oh, (((1,), (0,)), ((), ())),
                preferred_element_type=jnp.float32)
            return (idxv[0, :], q2, jnp.sum(oh, axis=1, keepdims=True))

        idx_j, qt_j, counts_j = lax.cond(tie, _slow, _fast)
        qt_ref[j] = xt + (qt_j - xt)
        idx_rows.append(idx_j)
        step_counts = counts_j if step_counts is None else step_counts + counts_j

    def _store_idx(rows_block):
        t = jnp.broadcast_to(rows_block.reshape(1, _BB, _K),
                             (_NB, _BB, _K)).reshape(_B, _K)
        rows = jax.lax.broadcasted_iota(jnp.int32, (_B, _K), 0)
        prev = jnp.where(i == 0, jnp.zeros((_B, _K), jnp.int32), idx_ref[...])
        idx_ref[...] = jnp.where((rows >> 3) == i, t, prev)

    _store_idx(jnp.stack(idx_rows, axis=0))

    @pl.when(i == 0)
    def _init():
        counts_ref[...] = step_counts

    @pl.when(i > 0)
    def _acc():
        counts_ref[...] += step_counts

    @pl.when(i == _NB - 1)
    def _final():
        p = counts_ref[...] * (1.0 / (_B * _R))
        ent = -jnp.sum(p * jnp.log(p + 1e-10))
        perp_ref[0, 0] = jnp.exp(ent)


def kernel(inputs, embeddings):
    xt = jnp.transpose(inputs, (0, 2, 1))
    et = embeddings.T
    qt, idx, perp = pl.pallas_call(
        _vq_body,
        grid=(_NB,),
        in_specs=[
            pl.BlockSpec((_BB, _D, _R), lambda i: (i, 0, 0)),
            pl.BlockSpec((_D, _K), lambda i: (0, 0)),
        ],
        out_specs=[
            pl.BlockSpec((_BB, _D, _R), lambda i: (i, 0, 0)),
            pl.BlockSpec((_B, _K), lambda i: (0, 0)),
            pl.BlockSpec(memory_space=pltpu.SMEM),
        ],
        out_shape=[
            jax.ShapeDtypeStruct((_B, _D, _R), jnp.float32),
            jax.ShapeDtypeStruct((_B, _K), jnp.int32),
            jax.ShapeDtypeStruct((1, 1), jnp.float32),
        ],
        scratch_shapes=[
            pltpu.VMEM((_G, _K), jnp.float32),
            pltpu.VMEM((_K, 1), jnp.float32),
        ],
    )(xt, et)
    quantized_st = jnp.transpose(qt, (0, 2, 1))
    return (quantized_st, idx, perp[0, 0])
